# Initial kernel scaffold; baseline (speedup 1.0000x reference)
import jax
import jax.numpy as jnp
from jax.experimental import pallas as pl


def kernel(x, edge_index, W1, b1, W2, b2, W3, b3, gamma, beta):
    raise NotImplementedError("write your pallas kernel here")


# speedup vs baseline: 4.7740x; 4.7740x over previous
import functools

import jax
import jax.numpy as jnp
from jax import lax
from jax.experimental import pallas as pl
from jax.experimental.pallas import tpu as pltpu
from jax.experimental.pallas import tpu_sc as plsc

N_NODES = 10000
D = 128
E_EDGES = 320000

NUM_CORES = 2
NUM_SUBCORES = 16
NW = NUM_CORES * NUM_SUBCORES

CHUNK = 128
CHUNKS = 79
E_PAD = NW * CHUNKS * CHUNK
N_PAD = 10240
ROWS_PER_TILE = N_PAD // NUM_SUBCORES
ZROWS = 64

_mesh = plsc.VectorSubcoreMesh(
    core_axis_name="c", subcore_axis_name="s",
    num_cores=NUM_CORES, num_subcores=NUM_SUBCORES)


@functools.partial(
    pl.kernel,
    out_type=jax.ShapeDtypeStruct((NUM_CORES * 2 * N_PAD,), jnp.float32),
    mesh=_mesh,
    scratch_types=[
        pltpu.VMEM((2 * CHUNKS, CHUNK), jnp.int32),
        pltpu.VMEM((CHUNK,), jnp.float32),
        pltpu.VMEM((2 * N_PAD // NUM_SUBCORES,), jnp.float32),
        pltpu.VMEM_SHARED((2 * N_PAD,), jnp.float32),
        pltpu.SemaphoreType.DMA,
    ],
)
def _sc_degrees(idx_hbm, out_hbm, idx_v, ones_v, z_v, acc, sem):
    c = lax.axis_index("c")
    s = lax.axis_index("s")
    wid = s * NUM_CORES + c

    ones16 = jnp.ones((16,), jnp.float32)
    zero16 = jnp.zeros((16,), jnp.float32)
    for i in range(CHUNK // 16):
        ones_v[pl.ds(i * 16, 16)] = ones16

    zlen = 2 * N_PAD // NUM_SUBCORES

    @pl.loop(0, zlen // 16)
    def _(i):
        z_v[pl.ds(i * 16, 16)] = zero16

    pltpu.sync_copy(z_v, acc.at[pl.ds(s * zlen, zlen)])
    plsc.subcore_barrier()

    pltpu.async_copy(idx_hbm.at[wid], idx_v, sem).wait()

    @pl.loop(0, 2 * CHUNKS)
    def _(j):
        pltpu.sync_copy(ones_v, acc.at[idx_v.at[j]], add=True)

    plsc.subcore_barrier()
    pltpu.sync_copy(acc.at[pl.ds(s * zlen, zlen)],
                    out_hbm.at[pl.ds(c * 2 * N_PAD + s * zlen, zlen)])


@functools.partial(
    pl.kernel,
    out_type=jax.ShapeDtypeStruct((NUM_CORES, N_PAD, D), jnp.float32),
    mesh=_mesh,
    scratch_types=[
        pltpu.VMEM((CHUNKS, CHUNK), jnp.int32),
        pltpu.VMEM((CHUNKS, CHUNK), jnp.int32),
        pltpu.VMEM((CHUNK, D), jnp.float32),
        pltpu.VMEM((ZROWS, D), jnp.float32),
        pltpu.VMEM_SHARED((N_PAD, D), jnp.float32),
        pltpu.SemaphoreType.DMA,
        pltpu.SemaphoreType.DMA,
    ],
)
def _sc_aggregate(g_hbm, src_hbm, dst_hbm, out_hbm,
                  src_v, dst_v, rows0, z_v, acc, sem0, isem):
    c = lax.axis_index("c")
    s = lax.axis_index("s")
    wid = s * NUM_CORES + c

    zero16 = jnp.zeros((16,), jnp.float32)

    @pl.loop(0, ZROWS)
    def _(r):
        for i in range(D // 16):
            z_v[r, pl.ds(i * 16, 16)] = zero16

    @pl.loop(0, ROWS_PER_TILE // ZROWS)
    def _(k):
        pltpu.sync_copy(z_v, acc.at[pl.ds(s * ROWS_PER_TILE + k * ZROWS,
                                          ZROWS)])

    pltpu.async_copy(src_hbm.at[wid], src_v, isem).wait()
    pltpu.async_copy(dst_hbm.at[wid], dst_v, isem).wait()
    plsc.subcore_barrier()

    @pl.loop(0, CHUNKS)
    def _(j):
        pltpu.async_copy(g_hbm.at[src_v.at[j]], rows0, sem0).wait()
        pltpu.sync_copy(rows0, acc.at[dst_v.at[j]], add=True)

    plsc.subcore_barrier()
    pltpu.sync_copy(acc.at[pl.ds(s * ROWS_PER_TILE, ROWS_PER_TILE)],
                    out_hbm.at[c, pl.ds(s * ROWS_PER_TILE, ROWS_PER_TILE)])


BLK = 1024


def _norm_from_deg(dega, degb):
    deg = dega + degb
    return jnp.where(deg > 0.0, lax.rsqrt(jnp.maximum(deg, 1.0)), 0.0)


def _tc_first_body(x_ref, doa_ref, dob_ref, dia_ref, dib_ref,
                   g_ref, no_ref, ni_ref):
    norm_out = _norm_from_deg(doa_ref[...], dob_ref[...])
    norm_in = _norm_from_deg(dia_ref[...], dib_ref[...])
    g_ref[...] = x_ref[...] * norm_out
    no_ref[...] = norm_out
    ni_ref[...] = norm_in


def _tc_first(x, doa, dob, dia, dib):
    grid = (N_PAD // BLK,)
    row = pl.BlockSpec((BLK, 1), lambda i: (i, 0))
    mat = pl.BlockSpec((BLK, D), lambda i: (i, 0))
    return pl.pallas_call(
        _tc_first_body,
        grid=grid,
        in_specs=[mat, row, row, row, row],
        out_specs=[mat, row, row],
        out_shape=[jax.ShapeDtypeStruct((N_PAD, D), jnp.float32),
                   jax.ShapeDtypeStruct((N_PAD, 1), jnp.float32),
                   jax.ShapeDtypeStruct((N_PAD, 1), jnp.float32)],
    )(x, doa, dob, dia, dib)


def _tc_mid_body(sa_ref, sb_ref, w_ref, b_ref, no_ref, ni_ref, g_ref):
    srow = sa_ref[...] + sb_ref[...]
    m = jnp.dot(srow, w_ref[...], preferred_element_type=jnp.float32,
                precision=lax.Precision.HIGHEST)
    no = no_ref[...]
    g_ref[...] = (no * ni_ref[...]) * m + no * b_ref[...]


def _tc_mid(sa, sb, w, b, no, ni):
    grid = (N_PAD // BLK,)
    row = pl.BlockSpec((BLK, 1), lambda i: (i, 0))
    mat = pl.BlockSpec((BLK, D), lambda i: (i, 0))
    wsp = pl.BlockSpec((D, D), lambda i: (0, 0))
    bsp = pl.BlockSpec((1, D), lambda i: (0, 0))
    return pl.pallas_call(
        _tc_mid_body,
        grid=grid,
        in_specs=[mat, mat, wsp, bsp, row, row],
        out_specs=mat,
        out_shape=jax.ShapeDtypeStruct((N_PAD, D), jnp.float32),
    )(sa, sb, w, b, no, ni)


def _tc_final_body(sa_ref, sb_ref, w_ref, b_ref, ni_ref, gam_ref, bet_ref,
                   o_ref):
    srow = sa_ref[...] + sb_ref[...]
    m = jnp.dot(srow, w_ref[...], preferred_element_type=jnp.float32,
                precision=lax.Precision.HIGHEST)
    t = ni_ref[...] * m + b_ref[...]
    mu = jnp.mean(t, axis=1, keepdims=True)
    cen = t - mu
    var = jnp.mean(cen * cen, axis=1, keepdims=True)
    hn = cen * lax.rsqrt(var + 1e-5)
    o_ref[...] = hn * gam_ref[...] + bet_ref[...]


def _tc_final(sa, sb, w, b, ni, gamma, beta):
    grid = (N_PAD // BLK,)
    row = pl.BlockSpec((BLK, 1), lambda i: (i, 0))
    mat = pl.BlockSpec((BLK, D), lambda i: (i, 0))
    wsp = pl.BlockSpec((D, D), lambda i: (0, 0))
    bsp = pl.BlockSpec((1, D), lambda i: (0, 0))
    return pl.pallas_call(
        _tc_final_body,
        grid=grid,
        in_specs=[mat, mat, wsp, bsp, row, bsp, bsp],
        out_specs=mat,
        out_shape=jax.ShapeDtypeStruct((N_PAD, D), jnp.float32),
    )(sa, sb, w, b, ni, gamma, beta)


def kernel(x, edge_index, W1, b1, W2, b2, W3, b3, gamma, beta):
    f32 = jnp.float32
    src = edge_index[0]
    dst = edge_index[1]

    pad = E_PAD - E_EDGES
    padv = jnp.full((pad,), N_NODES, jnp.int32)
    src_p = jnp.concatenate([src, padv])
    dst_p = jnp.concatenate([dst, padv])
    src3d = src_p.reshape(NW, CHUNKS, CHUNK)
    dst3d = dst_p.reshape(NW, CHUNKS, CHUNK)

    idx_deg = jnp.concatenate([src3d, dst3d + N_PAD], axis=1)

    x_pad = jnp.concatenate([x, jnp.zeros((N_PAD - N_NODES, D), f32)])

    degp = _sc_degrees(idx_deg).reshape(NUM_CORES, 2 * N_PAD)
    doa = degp[0, :N_PAD, None]
    dob = degp[1, :N_PAD, None]
    dia = degp[0, N_PAD:, None]
    dib = degp[1, N_PAD:, None]

    g0, no, ni = _tc_first(x_pad, doa, dob, dia, dib)

    b1r = b1.reshape(1, D)
    b2r = b2.reshape(1, D)
    b3r = b3.reshape(1, D)

    s1 = _sc_aggregate(g0, src3d, dst3d)
    g1 = _tc_mid(s1[0], s1[1], W1, b1r, no, ni)
    s2 = _sc_aggregate(g1, src3d, dst3d)
    g2 = _tc_mid(s2[0], s2[1], W2, b2r, no, ni)
    s3 = _sc_aggregate(g2, src3d, dst3d)
    out = _tc_final(s3[0], s3[1], W3, b3r, ni,
                    gamma.reshape(1, D), beta.reshape(1, D))
    return out[:N_NODES]


# speedup vs baseline: 5.0548x; 1.0588x over previous
import functools

import jax
import jax.numpy as jnp
from jax import lax
from jax.experimental import pallas as pl
from jax.experimental.pallas import tpu as pltpu
from jax.experimental.pallas import tpu_sc as plsc

N_NODES = 10000
D = 128
DH = 64
E_EDGES = 320000

NUM_CORES = 2
NUM_SUBCORES = 16
NW = NUM_CORES * NUM_SUBCORES

CHUNK = 128
CHUNKS = 160
NBUF = 5
NGROUPS = CHUNKS // NBUF
E_PAD = NUM_SUBCORES * CHUNKS * CHUNK
DCHUNKS = E_PAD // (NW * CHUNK)
N_PAD = 10240
ROWS_PER_TILE = N_PAD // NUM_SUBCORES
ZROWS = 16

_mesh = plsc.VectorSubcoreMesh(
    core_axis_name="c", subcore_axis_name="s",
    num_cores=NUM_CORES, num_subcores=NUM_SUBCORES)


@functools.partial(
    pl.kernel,
    out_type=jax.ShapeDtypeStruct((NUM_CORES * 2 * N_PAD,), jnp.float32),
    mesh=_mesh,
    scratch_types=[
        pltpu.VMEM((2 * DCHUNKS, CHUNK), jnp.int32),
        pltpu.VMEM((CHUNK,), jnp.float32),
        pltpu.VMEM((2 * N_PAD // NUM_SUBCORES,), jnp.float32),
        pltpu.VMEM_SHARED((2 * N_PAD,), jnp.float32),
        pltpu.SemaphoreType.DMA,
    ],
)
def _sc_degrees(idx_hbm, out_hbm, idx_v, ones_v, z_v, acc, sem):
    c = lax.axis_index("c")
    s = lax.axis_index("s")
    wid = s * NUM_CORES + c

    ones16 = jnp.ones((16,), jnp.float32)
    zero16 = jnp.zeros((16,), jnp.float32)
    for i in range(CHUNK // 16):
        ones_v[pl.ds(i * 16, 16)] = ones16

    zlen = 2 * N_PAD // NUM_SUBCORES

    @pl.loop(0, zlen // 16)
    def _(i):
        z_v[pl.ds(i * 16, 16)] = zero16

    pltpu.sync_copy(z_v, acc.at[pl.ds(s * zlen, zlen)])
    plsc.subcore_barrier()

    pltpu.async_copy(idx_hbm.at[wid], idx_v, sem).wait()

    @pl.loop(0, 2 * DCHUNKS)
    def _(j):
        pltpu.sync_copy(ones_v, acc.at[idx_v.at[j]], add=True)

    plsc.subcore_barrier()
    pltpu.sync_copy(acc.at[pl.ds(s * zlen, zlen)],
                    out_hbm.at[pl.ds(c * 2 * N_PAD + s * zlen, zlen)])


@functools.partial(
    pl.kernel,
    out_type=jax.ShapeDtypeStruct((NUM_CORES, N_PAD, DH), jnp.float32),
    mesh=_mesh,
    scratch_types=[
        pltpu.VMEM((CHUNKS, CHUNK), jnp.int32),
        pltpu.VMEM((CHUNKS, CHUNK), jnp.int32),
        [pltpu.VMEM((CHUNK, DH), jnp.float32)] * NBUF,
        pltpu.VMEM((ZROWS, DH), jnp.float32),
        pltpu.VMEM_SHARED((N_PAD, DH), jnp.float32),
        [pltpu.SemaphoreType.DMA] * NBUF,
        [pltpu.SemaphoreType.DMA] * NBUF,
        pltpu.SemaphoreType.DMA,
    ],
    compiler_params=pltpu.CompilerParams(use_tc_tiling_on_sc=False),
)
def _sc_aggregate(ga_hbm, gb_hbm, src_hbm, dst_hbm, out_hbm,
                  src_v, dst_v, rows, z_v, acc, gsem, ssem, isem):
    c = lax.axis_index("c")
    s = lax.axis_index("s")

    zero16 = jnp.zeros((16,), jnp.float32)

    @pl.loop(0, ZROWS)
    def _(r):
        for i in range(DH // 16):
            z_v[r, pl.ds(i * 16, 16)] = zero16

    @pl.loop(0, ROWS_PER_TILE // ZROWS)
    def _(k):
        pltpu.sync_copy(z_v, acc.at[pl.ds(s * ROWS_PER_TILE + k * ZROWS,
                                          ZROWS)])

    pltpu.async_copy(src_hbm.at[s], src_v, isem).wait()
    pltpu.async_copy(dst_hbm.at[s], dst_v, isem).wait()
    plsc.subcore_barrier()

    def pipeline(g_hbm):
        for b in range(NBUF):
            pltpu.async_copy(g_hbm.at[src_v.at[b]], rows[b], gsem[b])

        @pl.loop(0, NGROUPS)
        def _(t):
            base = t * NBUF
            for b in range(NBUF):
                pltpu.make_async_copy(g_hbm.at[src_v.at[base + b]], rows[b],
                                      gsem[b]).wait()
                pltpu.async_copy(rows[b], acc.at[dst_v.at[base + b]],
                                 ssem[b], add=True)
            for b in range(NBUF):
                @pl.when(t + 1 < NGROUPS)
                def _(b=b):
                    pltpu.make_async_copy(rows[b], acc.at[dst_v.at[base + b]],
                                          ssem[b]).wait()
                    pltpu.async_copy(g_hbm.at[src_v.at[base + NBUF + b]],
                                     rows[b], gsem[b])

        for b in range(NBUF):
            pltpu.make_async_copy(rows[b],
                                  acc.at[dst_v.at[CHUNKS - NBUF + b]],
                                  ssem[b]).wait()

    @pl.when(c == 0)
    def _():
        pipeline(ga_hbm)

    @pl.when(c == 1)
    def _():
        pipeline(gb_hbm)

    plsc.subcore_barrier()
    pltpu.sync_copy(acc.at[pl.ds(s * ROWS_PER_TILE, ROWS_PER_TILE)],
                    out_hbm.at[c, pl.ds(s * ROWS_PER_TILE, ROWS_PER_TILE)])


BLK = 1024


def _norm_from_deg(dega, degb):
    deg = dega + degb
    return jnp.where(deg > 0.0, lax.rsqrt(jnp.maximum(deg, 1.0)), 0.0)


def _tc_first_body(x_ref, doa_ref, dob_ref, dia_ref, dib_ref,
                   ga_ref, gb_ref, no_ref, ni_ref):
    norm_out = _norm_from_deg(doa_ref[...], dob_ref[...])
    norm_in = _norm_from_deg(dia_ref[...], dib_ref[...])
    g = x_ref[...] * norm_out
    ga_ref[...] = g[:, :DH]
    gb_ref[...] = g[:, DH:]
    no_ref[...] = norm_out
    ni_ref[...] = norm_in


def _tc_first(x, doa, dob, dia, dib):
    grid = (N_PAD // BLK,)
    row = pl.BlockSpec((BLK, 1), lambda i: (i, 0))
    mat = pl.BlockSpec((BLK, D), lambda i: (i, 0))
    half = pl.BlockSpec((BLK, DH), lambda i: (i, 0))
    return pl.pallas_call(
        _tc_first_body,
        grid=grid,
        in_specs=[mat, row, row, row, row],
        out_specs=[half, half, row, row],
        out_shape=[jax.ShapeDtypeStruct((N_PAD, DH), jnp.float32),
                   jax.ShapeDtypeStruct((N_PAD, DH), jnp.float32),
                   jax.ShapeDtypeStruct((N_PAD, 1), jnp.float32),
                   jax.ShapeDtypeStruct((N_PAD, 1), jnp.float32)],
    )(x, doa, dob, dia, dib)


def _tc_mid_body(sa_ref, sb_ref, wt_ref, wb_ref, b_ref, no_ref, ni_ref,
                 ga_ref, gb_ref):
    m = (jnp.dot(sa_ref[...], wt_ref[...],
                 preferred_element_type=jnp.float32,
                 precision=lax.Precision.HIGHEST)
         + jnp.dot(sb_ref[...], wb_ref[...],
                   preferred_element_type=jnp.float32,
                   precision=lax.Precision.HIGHEST))
    no = no_ref[...]
    g = (no * ni_ref[...]) * m + no * b_ref[...]
    ga_ref[...] = g[:, :DH]
    gb_ref[...] = g[:, DH:]


def _tc_mid(sa, sb, wt, wb, b, no, ni):
    grid = (N_PAD // BLK,)
    row = pl.BlockSpec((BLK, 1), lambda i: (i, 0))
    half = pl.BlockSpec((BLK, DH), lambda i: (i, 0))
    wsp = pl.BlockSpec((DH, D), lambda i: (0, 0))
    bsp = pl.BlockSpec((1, D), lambda i: (0, 0))
    return pl.pallas_call(
        _tc_mid_body,
        grid=grid,
        in_specs=[half, half, wsp, wsp, bsp, row, row],
        out_specs=[half, half],
        out_shape=[jax.ShapeDtypeStruct((N_PAD, DH), jnp.float32),
                   jax.ShapeDtypeStruct((N_PAD, DH), jnp.float32)],
    )(sa, sb, wt, wb, b, no, ni)


def _tc_final_body(sa_ref, sb_ref, wt_ref, wb_ref, b_ref, ni_ref, gam_ref,
                   bet_ref, o_ref):
    m = (jnp.dot(sa_ref[...], wt_ref[...],
                 preferred_element_type=jnp.float32,
                 precision=lax.Precision.HIGHEST)
         + jnp.dot(sb_ref[...], wb_ref[...],
                   preferred_element_type=jnp.float32,
                   precision=lax.Precision.HIGHEST))
    t = ni_ref[...] * m + b_ref[...]
    mu = jnp.mean(t, axis=1, keepdims=True)
    cen = t - mu
    var = jnp.mean(cen * cen, axis=1, keepdims=True)
    hn = cen * lax.rsqrt(var + 1e-5)
    o_ref[...] = hn * gam_ref[...] + bet_ref[...]


def _tc_final(sa, sb, wt, wb, b, ni, gamma, beta):
    grid = (N_PAD // BLK,)
    row = pl.BlockSpec((BLK, 1), lambda i: (i, 0))
    half = pl.BlockSpec((BLK, DH), lambda i: (i, 0))
    mat = pl.BlockSpec((BLK, D), lambda i: (i, 0))
    wsp = pl.BlockSpec((DH, D), lambda i: (0, 0))
    bsp = pl.BlockSpec((1, D), lambda i: (0, 0))
    return pl.pallas_call(
        _tc_final_body,
        grid=grid,
        in_specs=[half, half, wsp, wsp, bsp, row, bsp, bsp],
        out_specs=mat,
        out_shape=jax.ShapeDtypeStruct((N_PAD, D), jnp.float32),
    )(sa, sb, wt, wb, b, ni, gamma, beta)


def kernel(x, edge_index, W1, b1, W2, b2, W3, b3, gamma, beta):
    f32 = jnp.float32
    src = edge_index[0]
    dst = edge_index[1]

    pad = E_PAD - E_EDGES
    padv = jnp.full((pad,), N_NODES, jnp.int32)
    src_p = jnp.concatenate([src, padv])
    dst_p = jnp.concatenate([dst, padv])
    src3d = src_p.reshape(NUM_SUBCORES, CHUNKS, CHUNK)
    dst3d = dst_p.reshape(NUM_SUBCORES, CHUNKS, CHUNK)

    idx_deg = jnp.concatenate([src_p.reshape(NW, DCHUNKS, CHUNK),
                               dst_p.reshape(NW, DCHUNKS, CHUNK) + N_PAD],
                              axis=1)

    x_pad = jnp.concatenate([x, jnp.zeros((N_PAD - N_NODES, D), f32)])

    degp = _sc_degrees(idx_deg).reshape(NUM_CORES, 2 * N_PAD)
    doa = degp[0, :N_PAD, None]
    dob = degp[1, :N_PAD, None]
    dia = degp[0, N_PAD:, None]
    dib = degp[1, N_PAD:, None]

    ga0, gb0, no, ni = _tc_first(x_pad, doa, dob, dia, dib)

    b1r = b1.reshape(1, D)
    b2r = b2.reshape(1, D)
    b3r = b3.reshape(1, D)

    s1 = _sc_aggregate(ga0, gb0, src3d, dst3d)
    ga1, gb1 = _tc_mid(s1[0], s1[1], W1[:DH], W1[DH:], b1r, no, ni)
    s2 = _sc_aggregate(ga1, gb1, src3d, dst3d)
    ga2, gb2 = _tc_mid(s2[0], s2[1], W2[:DH], W2[DH:], b2r, no, ni)
    s3 = _sc_aggregate(ga2, gb2, src3d, dst3d)
    out = _tc_final(s3[0], s3[1], W3[:DH], W3[DH:], b3r, ni,
                    gamma.reshape(1, D), beta.reshape(1, D))
    return out[:N_NODES]


# speedup vs baseline: 7.6486x; 1.5131x over previous
import functools

import jax
import jax.numpy as jnp
from jax import lax
from jax.experimental import pallas as pl
from jax.experimental.pallas import tpu as pltpu
from jax.experimental.pallas import tpu_sc as plsc

N_NODES = 10000
D = 128
DH = 64
E_EDGES = 320000

NUM_CORES = 2
NUM_SUBCORES = 16
NW = NUM_CORES * NUM_SUBCORES

CHUNK = 128
CHUNKS = 160
NBUF = 2
BI = 40
NBLOCKS = CHUNKS // BI
NGROUPS = BI // NBUF
E_PAD = NUM_SUBCORES * CHUNKS * CHUNK
DCHUNKS = E_PAD // (NW * CHUNK)
N_PAD = 10240
ROWS_PER_TILE = N_PAD // NUM_SUBCORES
ZROWS = 16

_mesh = plsc.VectorSubcoreMesh(
    core_axis_name="c", subcore_axis_name="s",
    num_cores=NUM_CORES, num_subcores=NUM_SUBCORES)


@functools.partial(
    pl.kernel,
    out_type=jax.ShapeDtypeStruct((NUM_CORES * 2 * N_PAD,), jnp.float32),
    mesh=_mesh,
    scratch_types=[
        pltpu.VMEM((2 * DCHUNKS, CHUNK), jnp.int32),
        pltpu.VMEM((CHUNK,), jnp.float32),
        pltpu.VMEM((2 * N_PAD // NUM_SUBCORES,), jnp.float32),
        pltpu.VMEM_SHARED((2 * N_PAD,), jnp.float32),
        pltpu.SemaphoreType.DMA,
    ],
)
def _sc_degrees(idx_hbm, out_hbm, idx_v, ones_v, z_v, acc, sem):
    c = lax.axis_index("c")
    s = lax.axis_index("s")
    wid = s * NUM_CORES + c

    ones16 = jnp.ones((16,), jnp.float32)
    zero16 = jnp.zeros((16,), jnp.float32)
    for i in range(CHUNK // 16):
        ones_v[pl.ds(i * 16, 16)] = ones16

    zlen = 2 * N_PAD // NUM_SUBCORES

    @pl.loop(0, zlen // 16)
    def _(i):
        z_v[pl.ds(i * 16, 16)] = zero16

    pltpu.sync_copy(z_v, acc.at[pl.ds(s * zlen, zlen)])
    plsc.subcore_barrier()

    pltpu.async_copy(idx_hbm.at[wid], idx_v, sem).wait()

    @pl.loop(0, 2 * DCHUNKS)
    def _(j):
        pltpu.sync_copy(ones_v, acc.at[idx_v.at[j]], add=True)

    plsc.subcore_barrier()
    pltpu.sync_copy(acc.at[pl.ds(s * zlen, zlen)],
                    out_hbm.at[pl.ds(c * 2 * N_PAD + s * zlen, zlen)])


@functools.partial(
    pl.kernel,
    out_type=jax.ShapeDtypeStruct((NUM_CORES, N_PAD, DH), jnp.float32),
    mesh=_mesh,
    scratch_types=[
        pltpu.VMEM((2, BI, CHUNK), jnp.int32),
        pltpu.VMEM((2, BI, CHUNK), jnp.int32),
        [pltpu.VMEM((CHUNK, DH), jnp.float32)] * NBUF,
        pltpu.VMEM((ZROWS, DH), jnp.float32),
        pltpu.VMEM_SHARED((N_PAD, DH), jnp.float32),
        pltpu.VMEM_SHARED((N_PAD, DH), jnp.float32),
        [pltpu.SemaphoreType.DMA] * NBUF,
        [pltpu.SemaphoreType.DMA] * NBUF,
        [pltpu.SemaphoreType.DMA] * 2,
        [pltpu.SemaphoreType.DMA] * 2,
    ],
    compiler_params=pltpu.CompilerParams(use_tc_tiling_on_sc=False),
)
def _sc_aggregate(ga_hbm, gb_hbm, src_hbm, dst_hbm, out_hbm,
                  src_v, dst_v, rows, z_v, gtab, acc, gsem, ssem, bs_sem,
                  bd_sem):
    c = lax.axis_index("c")
    s = lax.axis_index("s")

    zero16 = jnp.zeros((16,), jnp.float32)

    @pl.loop(0, ZROWS)
    def _(r):
        for i in range(DH // 16):
            z_v[r, pl.ds(i * 16, 16)] = zero16

    @pl.loop(0, ROWS_PER_TILE // ZROWS)
    def _(k):
        pltpu.sync_copy(z_v, acc.at[pl.ds(s * ROWS_PER_TILE + k * ZROWS,
                                          ZROWS)])

    rslice = pl.ds(s * ROWS_PER_TILE, ROWS_PER_TILE)

    @pl.when(c == 0)
    def _():
        pltpu.sync_copy(ga_hbm.at[rslice], gtab.at[rslice])

    @pl.when(c == 1)
    def _():
        pltpu.sync_copy(gb_hbm.at[rslice], gtab.at[rslice])

    pltpu.async_copy(src_hbm.at[s, pl.ds(0, BI)], src_v.at[0], bs_sem[0])
    pltpu.async_copy(dst_hbm.at[s, pl.ds(0, BI)], dst_v.at[0], bd_sem[0])
    plsc.subcore_barrier()

    for k in range(NBLOCKS):
        slot = k % 2
        pltpu.make_async_copy(src_hbm.at[s, pl.ds(k * BI, BI)],
                              src_v.at[slot], bs_sem[slot]).wait()
        pltpu.make_async_copy(dst_hbm.at[s, pl.ds(k * BI, BI)],
                              dst_v.at[slot], bd_sem[slot]).wait()
        if k + 1 < NBLOCKS:
            nslot = (k + 1) % 2
            pltpu.async_copy(src_hbm.at[s, pl.ds((k + 1) * BI, BI)],
                             src_v.at[nslot], bs_sem[nslot])
            pltpu.async_copy(dst_hbm.at[s, pl.ds((k + 1) * BI, BI)],
                             dst_v.at[nslot], bd_sem[nslot])

        for b in range(NBUF):
            pltpu.async_copy(gtab.at[src_v.at[slot, b]], rows[b], gsem[b])

        @pl.loop(0, NGROUPS)
        def _(t, slot=slot):
            base = t * NBUF
            for b in range(NBUF):
                pltpu.make_async_copy(gtab.at[src_v.at[slot, base + b]],
                                      rows[b], gsem[b]).wait()
                pltpu.async_copy(rows[b], acc.at[dst_v.at[slot, base + b]],
                                 ssem[b], add=True)
            for b in range(NBUF):
                @pl.when(t + 1 < NGROUPS)
                def _(b=b):
                    pltpu.make_async_copy(rows[b],
                                          acc.at[dst_v.at[slot, base + b]],
                                          ssem[b]).wait()
                    pltpu.async_copy(gtab.at[src_v.at[slot, base + NBUF + b]],
                                     rows[b], gsem[b])

        for b in range(NBUF):
            pltpu.make_async_copy(rows[b],
                                  acc.at[dst_v.at[slot, BI - NBUF + b]],
                                  ssem[b]).wait()

    plsc.subcore_barrier()
    pltpu.sync_copy(acc.at[pl.ds(s * ROWS_PER_TILE, ROWS_PER_TILE)],
                    out_hbm.at[c, pl.ds(s * ROWS_PER_TILE, ROWS_PER_TILE)])


BLK = 1024


def _norm_from_deg(dega, degb):
    deg = dega + degb
    return jnp.where(deg > 0.0, lax.rsqrt(jnp.maximum(deg, 1.0)), 0.0)


def _tc_first_body(x_ref, doa_ref, dob_ref, dia_ref, dib_ref,
                   ga_ref, gb_ref, no_ref, ni_ref):
    norm_out = _norm_from_deg(doa_ref[...], dob_ref[...])
    norm_in = _norm_from_deg(dia_ref[...], dib_ref[...])
    g = x_ref[...] * norm_out
    ga_ref[...] = g[:, :DH]
    gb_ref[...] = g[:, DH:]
    no_ref[...] = norm_out
    ni_ref[...] = norm_in


def _tc_first(x, doa, dob, dia, dib):
    grid = (N_PAD // BLK,)
    row = pl.BlockSpec((BLK, 1), lambda i: (i, 0))
    mat = pl.BlockSpec((BLK, D), lambda i: (i, 0))
    half = pl.BlockSpec((BLK, DH), lambda i: (i, 0))
    return pl.pallas_call(
        _tc_first_body,
        grid=grid,
        in_specs=[mat, row, row, row, row],
        out_specs=[half, half, row, row],
        out_shape=[jax.ShapeDtypeStruct((N_PAD, DH), jnp.float32),
                   jax.ShapeDtypeStruct((N_PAD, DH), jnp.float32),
                   jax.ShapeDtypeStruct((N_PAD, 1), jnp.float32),
                   jax.ShapeDtypeStruct((N_PAD, 1), jnp.float32)],
    )(x, doa, dob, dia, dib)


def _tc_mid_body(sa_ref, sb_ref, wt_ref, wb_ref, b_ref, no_ref, ni_ref,
                 ga_ref, gb_ref):
    m = (jnp.dot(sa_ref[...], wt_ref[...],
                 preferred_element_type=jnp.float32,
                 precision=lax.Precision.HIGHEST)
         + jnp.dot(sb_ref[...], wb_ref[...],
                   preferred_element_type=jnp.float32,
                   precision=lax.Precision.HIGHEST))
    no = no_ref[...]
    g = (no * ni_ref[...]) * m + no * b_ref[...]
    ga_ref[...] = g[:, :DH]
    gb_ref[...] = g[:, DH:]


def _tc_mid(sa, sb, wt, wb, b, no, ni):
    grid = (N_PAD // BLK,)
    row = pl.BlockSpec((BLK, 1), lambda i: (i, 0))
    half = pl.BlockSpec((BLK, DH), lambda i: (i, 0))
    wsp = pl.BlockSpec((DH, D), lambda i: (0, 0))
    bsp = pl.BlockSpec((1, D), lambda i: (0, 0))
    return pl.pallas_call(
        _tc_mid_body,
        grid=grid,
        in_specs=[half, half, wsp, wsp, bsp, row, row],
        out_specs=[half, half],
        out_shape=[jax.ShapeDtypeStruct((N_PAD, DH), jnp.float32),
                   jax.ShapeDtypeStruct((N_PAD, DH), jnp.float32)],
    )(sa, sb, wt, wb, b, no, ni)


def _tc_final_body(sa_ref, sb_ref, wt_ref, wb_ref, b_ref, ni_ref, gam_ref,
                   bet_ref, o_ref):
    m = (jnp.dot(sa_ref[...], wt_ref[...],
                 preferred_element_type=jnp.float32,
                 precision=lax.Precision.HIGHEST)
         + jnp.dot(sb_ref[...], wb_ref[...],
                   preferred_element_type=jnp.float32,
                   precision=lax.Precision.HIGHEST))
    t = ni_ref[...] * m + b_ref[...]
    mu = jnp.mean(t, axis=1, keepdims=True)
    cen = t - mu
    var = jnp.mean(cen * cen, axis=1, keepdims=True)
    hn = cen * lax.rsqrt(var + 1e-5)
    o_ref[...] = hn * gam_ref[...] + bet_ref[...]


def _tc_final(sa, sb, wt, wb, b, ni, gamma, beta):
    grid = (N_PAD // BLK,)
    row = pl.BlockSpec((BLK, 1), lambda i: (i, 0))
    half = pl.BlockSpec((BLK, DH), lambda i: (i, 0))
    mat = pl.BlockSpec((BLK, D), lambda i: (i, 0))
    wsp = pl.BlockSpec((DH, D), lambda i: (0, 0))
    bsp = pl.BlockSpec((1, D), lambda i: (0, 0))
    return pl.pallas_call(
        _tc_final_body,
        grid=grid,
        in_specs=[half, half, wsp, wsp, bsp, row, bsp, bsp],
        out_specs=mat,
        out_shape=jax.ShapeDtypeStruct((N_PAD, D), jnp.float32),
    )(sa, sb, wt, wb, b, ni, gamma, beta)


def kernel(x, edge_index, W1, b1, W2, b2, W3, b3, gamma, beta):
    f32 = jnp.float32
    src = edge_index[0]
    dst = edge_index[1]

    pad = E_PAD - E_EDGES
    padv = jnp.full((pad,), N_NODES, jnp.int32)
    src_p = jnp.concatenate([src, padv])
    dst_p = jnp.concatenate([dst, padv])
    src3d = src_p.reshape(NUM_SUBCORES, CHUNKS, CHUNK)
    dst3d = dst_p.reshape(NUM_SUBCORES, CHUNKS, CHUNK)

    idx_deg = jnp.concatenate([src_p.reshape(NW, DCHUNKS, CHUNK),
                               dst_p.reshape(NW, DCHUNKS, CHUNK) + N_PAD],
                              axis=1)

    x_pad = jnp.concatenate([x, jnp.zeros((N_PAD - N_NODES, D), f32)])

    degp = _sc_degrees(idx_deg).reshape(NUM_CORES, 2 * N_PAD)
    doa = degp[0, :N_PAD, None]
    dob = degp[1, :N_PAD, None]
    dia = degp[0, N_PAD:, None]
    dib = degp[1, N_PAD:, None]

    ga0, gb0, no, ni = _tc_first(x_pad, doa, dob, dia, dib)

    b1r = b1.reshape(1, D)
    b2r = b2.reshape(1, D)
    b3r = b3.reshape(1, D)

    s1 = _sc_aggregate(ga0, gb0, src3d, dst3d)
    ga1, gb1 = _tc_mid(s1[0], s1[1], W1[:DH], W1[DH:], b1r, no, ni)
    s2 = _sc_aggregate(ga1, gb1, src3d, dst3d)
    ga2, gb2 = _tc_mid(s2[0], s2[1], W2[:DH], W2[DH:], b2r, no, ni)
    s3 = _sc_aggregate(ga2, gb2, src3d, dst3d)
    out = _tc_final(s3[0], s3[1], W3[:DH], W3[DH:], b3r, ni,
                    gamma.reshape(1, D), beta.reshape(1, D))
    return out[:N_NODES]
